# baseline (speedup 1.0000x reference)
import functools

import jax
import jax.numpy as jnp
from jax import lax
from jax.experimental import pallas as pl
from jax.experimental.pallas import tpu as pltpu

SCORE_THRESH = 0.7
TOKEN_THRESH = 0.08
NUM_CLASSES = 256
ROWS = 5000
BATCH = 16
BLOCK_ROWS = 5000


def _post_kernel(ts_ref, logits_ref, boxes_ref,
                 scores_ref, labels_ref, boxes_out_ref, keep_ref,
                 xywh_ref, pos_ref):
    b = pl.program_id(0)

    x = logits_ref[0]
    m = jnp.max(x, axis=-1, keepdims=True)
    e = jnp.exp(x - m)
    s = jnp.sum(e, axis=-1, keepdims=True)

    pos_ref[0] = e > (TOKEN_THRESH * s)

    col = lax.broadcasted_iota(jnp.int32, x.shape, 1)
    valid = col < (NUM_CLASSES - 1)
    emax = jnp.max(jnp.where(valid, e, 0.0), axis=-1)
    scores = 1.0 - emax / s[:, 0]
    scores_ref[0, 0] = scores
    keep_ref[0, 0] = scores > SCORE_THRESH

    xm = jnp.where(valid, x, -jnp.inf)
    xmax = jnp.max(xm, axis=-1, keepdims=True)
    idx = jnp.where(xm == xmax, col, NUM_CLASSES)
    labels_ref[0, 0] = jnp.min(idx, axis=-1).astype(jnp.int32)

    bx = boxes_ref[0]
    half_wh = 0.5 * bx[:, 2:]
    xy0 = bx[:, :2] - half_wh
    xy1 = bx[:, :2] + half_wh
    xyxy = jnp.concatenate([xy0, xy1], axis=-1)
    img_h = ts_ref[b, 0].astype(jnp.float32)
    img_w = ts_ref[b, 1].astype(jnp.float32)
    col4 = lax.broadcasted_iota(jnp.int32, xyxy.shape, 1)
    scale = jnp.where((col4 % 2) == 0, img_w, img_h)
    sb = xyxy * scale
    boxes_out_ref[0] = sb
    xywh_ref[0] = jnp.concatenate([sb[:, :2], sb[:, 2:] - sb[:, :2]], axis=-1)


@functools.partial(jax.jit, static_argnames=())
def kernel(pred_logits, pred_boxes, target_sizes):
    grid = (BATCH,)

    ts = target_sizes.astype(jnp.int32)

    out_shapes = (
        jax.ShapeDtypeStruct((BATCH, 1, ROWS), jnp.float32),
        jax.ShapeDtypeStruct((BATCH, 1, ROWS), jnp.int32),
        jax.ShapeDtypeStruct((BATCH, ROWS, 4), jnp.float32),
        jax.ShapeDtypeStruct((BATCH, 1, ROWS), jnp.bool_),
        jax.ShapeDtypeStruct((BATCH, ROWS, 4), jnp.float32),
        jax.ShapeDtypeStruct((BATCH, ROWS, NUM_CLASSES), jnp.bool_),
    )

    row_map = lambda b: (b, 0, 0)
    vec_map = lambda b: (b, 0, 0)

    scores3, labels3, boxes, keep3, xywh, pos = pl.pallas_call(
        _post_kernel,
        grid=grid,
        in_specs=[
            pl.BlockSpec(memory_space=pltpu.SMEM),
            pl.BlockSpec((1, BLOCK_ROWS, NUM_CLASSES), row_map),
            pl.BlockSpec((1, BLOCK_ROWS, 4), row_map),
        ],
        out_specs=(
            pl.BlockSpec((1, 1, BLOCK_ROWS), vec_map),
            pl.BlockSpec((1, 1, BLOCK_ROWS), vec_map),
            pl.BlockSpec((1, BLOCK_ROWS, 4), row_map),
            pl.BlockSpec((1, 1, BLOCK_ROWS), vec_map),
            pl.BlockSpec((1, BLOCK_ROWS, 4), row_map),
            pl.BlockSpec((1, BLOCK_ROWS, NUM_CLASSES), row_map),
        ),
        out_shape=out_shapes,
    )(ts, pred_logits, pred_boxes)

    scores = scores3.reshape(BATCH, ROWS)
    labels = labels3.reshape(BATCH, ROWS)
    keep = keep3.reshape(BATCH, ROWS)
    return (scores, labels, boxes, keep, xywh, pos)


# speedup vs baseline: 1.3232x; 1.3232x over previous
import jax
import jax.numpy as jnp
from jax import lax
from jax.experimental import pallas as pl
from jax.experimental.pallas import tpu as pltpu

SCORE_THRESH = 0.7
TOKEN_THRESH = 0.08
NUM_CLASSES = 256
ROWS = 5000
BATCH = 16


def _post_kernel(ts_ref, logits_ref, boxes_ref,
                 scores_ref, labels_ref, boxes_out_ref, keep_ref,
                 xywh_ref, pos_ref):
    b = pl.program_id(0)

    x = logits_ref[0]
    xt = x.T
    m = jnp.max(xt, axis=0, keepdims=True)
    e = jnp.exp(xt - m)
    s = jnp.sum(e, axis=0, keepdims=True)

    pos_ref[0] = e > (TOKEN_THRESH * s)

    row = lax.broadcasted_iota(jnp.int32, xt.shape, 0)
    valid = row < (NUM_CLASSES - 1)
    em = jnp.where(valid, e, 0.0)
    emax = jnp.max(em, axis=0)
    scores = 1.0 - emax / s[0]
    scores_ref[0, 0] = scores
    keep_ref[0, 0] = scores > SCORE_THRESH

    idx = jnp.where(em == emax[None, :], row, NUM_CLASSES)
    labels_ref[0, 0] = jnp.min(idx, axis=0).astype(jnp.int32)

    bx = boxes_ref[0]
    half_wh = 0.5 * bx[:, 2:]
    xy0 = bx[:, :2] - half_wh
    xy1 = bx[:, :2] + half_wh
    xyxy = jnp.concatenate([xy0, xy1], axis=-1)
    img_h = ts_ref[b, 0].astype(jnp.float32)
    img_w = ts_ref[b, 1].astype(jnp.float32)
    col4 = lax.broadcasted_iota(jnp.int32, xyxy.shape, 1)
    scale = jnp.where((col4 % 2) == 0, img_w, img_h)
    sb = xyxy * scale
    boxes_out_ref[0] = sb
    xywh_ref[0] = jnp.concatenate([sb[:, :2], sb[:, 2:] - sb[:, :2]], axis=-1)


@jax.jit
def kernel(pred_logits, pred_boxes, target_sizes):
    grid = (BATCH,)
    ts = target_sizes.astype(jnp.int32)

    out_shapes = (
        jax.ShapeDtypeStruct((BATCH, 1, ROWS), jnp.float32),
        jax.ShapeDtypeStruct((BATCH, 1, ROWS), jnp.int32),
        jax.ShapeDtypeStruct((BATCH, ROWS, 4), jnp.float32),
        jax.ShapeDtypeStruct((BATCH, 1, ROWS), jnp.bool_),
        jax.ShapeDtypeStruct((BATCH, ROWS, 4), jnp.float32),
        jax.ShapeDtypeStruct((BATCH, NUM_CLASSES, ROWS), jnp.bool_),
    )

    bmap = lambda b: (b, 0, 0)

    scores3, labels3, boxes, keep3, xywh, pos_t = pl.pallas_call(
        _post_kernel,
        grid=grid,
        in_specs=[
            pl.BlockSpec(memory_space=pltpu.SMEM),
            pl.BlockSpec((1, ROWS, NUM_CLASSES), bmap),
            pl.BlockSpec((1, ROWS, 4), bmap),
        ],
        out_specs=(
            pl.BlockSpec((1, 1, ROWS), bmap),
            pl.BlockSpec((1, 1, ROWS), bmap),
            pl.BlockSpec((1, ROWS, 4), bmap),
            pl.BlockSpec((1, 1, ROWS), bmap),
            pl.BlockSpec((1, ROWS, 4), bmap),
            pl.BlockSpec((1, NUM_CLASSES, ROWS), bmap),
        ),
        out_shape=out_shapes,
        compiler_params=pltpu.CompilerParams(
            dimension_semantics=("parallel",),
        ),
    )(ts, pred_logits, pred_boxes)

    scores = scores3.reshape(BATCH, ROWS)
    labels = labels3.reshape(BATCH, ROWS)
    keep = keep3.reshape(BATCH, ROWS)
    pos = jnp.transpose(pos_t, (0, 2, 1))
    return (scores, labels, boxes, keep, xywh, pos)


# speedup vs baseline: 1.4112x; 1.0665x over previous
import jax
import jax.numpy as jnp
from jax import lax
from jax.experimental import pallas as pl
from jax.experimental.pallas import tpu as pltpu

SCORE_THRESH = 0.7
TOKEN_THRESH = 0.08
NUM_CLASSES = 256
ROWS = 5000
BATCH = 16


def _post_kernel(ts_ref, logits_ref, boxes_ref,
                 scores_ref, labels_ref, boxes_out_ref, keep_ref,
                 xywh_ref, pos_ref):
    b = pl.program_id(0)

    x = logits_ref[0]
    xt = x.T
    m = jnp.max(xt, axis=0, keepdims=True)
    e = jnp.exp(xt - m)
    s = jnp.sum(e, axis=0, keepdims=True)

    st = jnp.concatenate([m, s, m, s, m, s, m, s], axis=0)
    stT = st.T
    m_col = stT[:, 0:1]
    s_col = stT[:, 1:2]

    e_o = jnp.exp(x - m_col)
    pos_ref[0] = e_o > (TOKEN_THRESH * s_col)

    row = lax.broadcasted_iota(jnp.int32, xt.shape, 0)
    valid = row < (NUM_CLASSES - 1)
    em = jnp.where(valid, e, 0.0)
    emax = jnp.max(em, axis=0)
    scores = 1.0 - emax / s[0]
    scores_ref[0, 0] = scores
    keep_ref[0, 0] = scores > SCORE_THRESH

    idx = jnp.where(em == emax[None, :], row, NUM_CLASSES)
    labels_ref[0, 0] = jnp.min(idx, axis=0).astype(jnp.int32)

    bx = boxes_ref[0]
    half_wh = 0.5 * bx[:, 2:]
    xy0 = bx[:, :2] - half_wh
    xy1 = bx[:, :2] + half_wh
    xyxy = jnp.concatenate([xy0, xy1], axis=-1)
    img_h = ts_ref[b, 0].astype(jnp.float32)
    img_w = ts_ref[b, 1].astype(jnp.float32)
    col4 = lax.broadcasted_iota(jnp.int32, xyxy.shape, 1)
    scale = jnp.where((col4 % 2) == 0, img_w, img_h)
    sb = xyxy * scale
    boxes_out_ref[0] = sb
    xywh_ref[0] = jnp.concatenate([sb[:, :2], sb[:, 2:] - sb[:, :2]], axis=-1)


@jax.jit
def kernel(pred_logits, pred_boxes, target_sizes):
    grid = (BATCH,)
    ts = target_sizes.astype(jnp.int32)

    out_shapes = (
        jax.ShapeDtypeStruct((BATCH, 1, ROWS), jnp.float32),
        jax.ShapeDtypeStruct((BATCH, 1, ROWS), jnp.int32),
        jax.ShapeDtypeStruct((BATCH, ROWS, 4), jnp.float32),
        jax.ShapeDtypeStruct((BATCH, 1, ROWS), jnp.bool_),
        jax.ShapeDtypeStruct((BATCH, ROWS, 4), jnp.float32),
        jax.ShapeDtypeStruct((BATCH, ROWS, NUM_CLASSES), jnp.bool_),
    )

    bmap = lambda b: (b, 0, 0)

    scores3, labels3, boxes, keep3, xywh, pos = pl.pallas_call(
        _post_kernel,
        grid=grid,
        in_specs=[
            pl.BlockSpec(memory_space=pltpu.SMEM),
            pl.BlockSpec((1, ROWS, NUM_CLASSES), bmap),
            pl.BlockSpec((1, ROWS, 4), bmap),
        ],
        out_specs=(
            pl.BlockSpec((1, 1, ROWS), bmap),
            pl.BlockSpec((1, 1, ROWS), bmap),
            pl.BlockSpec((1, ROWS, 4), bmap),
            pl.BlockSpec((1, 1, ROWS), bmap),
            pl.BlockSpec((1, ROWS, 4), bmap),
            pl.BlockSpec((1, ROWS, NUM_CLASSES), bmap),
        ),
        out_shape=out_shapes,
        compiler_params=pltpu.CompilerParams(
            dimension_semantics=("parallel",),
        ),
    )(ts, pred_logits, pred_boxes)

    scores = scores3.reshape(BATCH, ROWS)
    labels = labels3.reshape(BATCH, ROWS)
    keep = keep3.reshape(BATCH, ROWS)
    return (scores, labels, boxes, keep, xywh, pos)
